# baseline (speedup 1.0000x reference)
import jax, jax.numpy as jnp

N = 10000

def _gat(x, src, dst, W, a_src, a_dst, bias, heads, feat, concat):
    n = x.shape[0]
    h = (x @ W).reshape(n, heads, feat)
    asrc = (h * a_src[None]).sum(-1)
    adst = (h * a_dst[None]).sum(-1)
    e = jax.nn.leaky_relu(asrc[src] + adst[dst], 0.2)
    m = jax.ops.segment_max(e, dst, num_segments=n)
    ex = jnp.exp(e - m[dst])
    s = jax.ops.segment_sum(ex, dst, num_segments=n)
    alpha = ex / (s[dst] + 1e-16)
    out = jax.ops.segment_sum(h[src] * alpha[:, :, None], dst, num_segments=n)
    if concat:
        return out.reshape(n, heads * feat) + bias
    return out.mean(axis=1) + bias


def kernel(x, edge_index, W1, a_src1, a_dst1, b1, W2, a_src2, a_dst2, b2):
    n = x.shape[0]
    src = edge_index[0]; dst = edge_index[1]
    loop = jnp.arange(n, dtype=src.dtype)
    src = jnp.concatenate([src, loop])
    dst = jnp.concatenate([dst, loop])
    h = jax.nn.elu(_gat(x, src, dst, W1, a_src1, a_dst1, b1, 8, 8, True))
    return _gat(h, src, dst, W2, a_src2, a_dst2, b2, 8, 8, False)


# speedup vs baseline: 2.6268x; 2.6268x over previous
import functools

import numpy as np
import jax
import jax.numpy as jnp
from jax import lax
from jax.experimental import pallas as pl
from jax.experimental.pallas import tpu as pltpu
from jax.experimental.pallas import tpu_sc as plsc

N = 10000
E = 320000
IN = 128
HID = 8
OUT = 8
HEADS = 8

NC = 2
NS = 16
NW = NC * NS
NPAD = 10240
RANGE = NPAD // NW
ROWW = 80
TABW = 128
AW = 16
ETOT = E + N
ECH = 2048
EPAD = 331776
NCHUNK = EPAD // ECH
TRASH = RANGE

RB = 1280
GRID = NPAD // RB

_PERM = np.array([(c % 8) * 8 + c // 8 for c in range(64)])


def _tc_prep_body(x_ref, tw_ref, aw_ref, tab_ref, adt_ref):
    xv = x_ref[...]
    tab_ref[...] = jnp.dot(xv, tw_ref[...], preferred_element_type=jnp.float32)
    adt_ref[...] = jnp.dot(xv, aw_ref[...], preferred_element_type=jnp.float32)


def _tc_mid_body(acc_ref, b1_ref, selk_ref, tw_ref, aw_ref, tab_ref, adt_ref):
    acc = acc_ref[...]
    recs = 1.0 / (acc[:, 64:72] + 1e-16)
    div64 = jnp.dot(recs, selk_ref[...], preferred_element_type=jnp.float32)
    h1 = acc[:, 0:64] * div64 + b1_ref[...]
    h1 = jnp.where(h1 > 0, h1, jnp.exp(h1) - 1.0)
    tab_ref[...] = jnp.dot(h1, tw_ref[...], preferred_element_type=jnp.float32)
    adt_ref[...] = jnp.dot(h1, aw_ref[...], preferred_element_type=jnp.float32)


def _tc_fin_body(acc_ref, b2_ref, selk_ref, m3_ref, out_ref):
    acc = acc_ref[...]
    recs = 1.0 / (acc[:, 64:72] + 1e-16)
    div64 = jnp.dot(recs, selk_ref[...], preferred_element_type=jnp.float32)
    out_ref[...] = (jnp.dot(acc[:, 0:64] * div64, m3_ref[...],
                            preferred_element_type=jnp.float32) + b2_ref[...])


def _full(shape):
    return pl.BlockSpec(shape, lambda i: (0,) * len(shape))


_tc_prep = pl.pallas_call(
    _tc_prep_body,
    grid=(GRID,),
    in_specs=[pl.BlockSpec((RB, IN), lambda i: (i, 0)),
              _full((IN, TABW)), _full((IN, AW))],
    out_specs=[pl.BlockSpec((RB, TABW), lambda i: (i, 0)),
               pl.BlockSpec((RB, AW), lambda i: (i, 0))],
    out_shape=[jax.ShapeDtypeStruct((NPAD, TABW), jnp.float32),
               jax.ShapeDtypeStruct((NPAD, AW), jnp.float32)],
)

_tc_mid = pl.pallas_call(
    _tc_mid_body,
    grid=(GRID,),
    in_specs=[pl.BlockSpec((RB, ROWW), lambda i: (i, 0)),
              _full((1, 64)), _full((8, 64)),
              _full((64, TABW)), _full((64, AW))],
    out_specs=[pl.BlockSpec((RB, TABW), lambda i: (i, 0)),
               pl.BlockSpec((RB, AW), lambda i: (i, 0))],
    out_shape=[jax.ShapeDtypeStruct((NPAD, TABW), jnp.float32),
               jax.ShapeDtypeStruct((NPAD, AW), jnp.float32)],
)

_tc_fin = pl.pallas_call(
    _tc_fin_body,
    grid=(GRID,),
    in_specs=[pl.BlockSpec((RB, ROWW), lambda i: (i, 0)),
              _full((1, OUT)), _full((8, 64)), _full((64, OUT))],
    out_specs=pl.BlockSpec((RB, OUT), lambda i: (i, 0)),
    out_shape=jax.ShapeDtypeStruct((NPAD, OUT), jnp.float32),
)


_mesh = plsc.VectorSubcoreMesh(core_axis_name="c", subcore_axis_name="s",
                               num_cores=NC, num_subcores=NS)


@functools.partial(
    pl.kernel,
    out_type=jax.ShapeDtypeStruct((NPAD, ROWW), jnp.float32),
    mesh=_mesh,
    scratch_types=[
        pltpu.VMEM((ECH,), jnp.int32),
        pltpu.VMEM((ECH,), jnp.int32),
        pltpu.VMEM((RANGE + 8, AW), jnp.float32),
        pltpu.VMEM((RANGE + 8, ROWW), jnp.float32),
        pltpu.VMEM((2, 16, TABW), jnp.float32),
        pltpu.VMEM((2, 16), jnp.int32),
        pltpu.VMEM((16,), jnp.int32),
        pltpu.VMEM((32,), jnp.int32),
        pltpu.SMEM((2, 16), jnp.int32),
        pltpu.SMEM((1,), jnp.int32),
        pltpu.SemaphoreType.DMA,
    ],
)
def _edge_pass(tab_hbm, adt_hbm, src_hbm, dst_hbm, out_hbm,
               dch, sch, adl, acc, rows2, gidx, pend, fbuf,
               sdl, scnt, sem):
    cid = lax.axis_index("c")
    sid = lax.axis_index("s")
    wid = cid * NS + sid
    lo = wid * RANGE
    z16 = jnp.zeros((16,), jnp.float32)
    z16i = jnp.zeros((16,), jnp.int32)
    one16i = jnp.full((16,), 1, jnp.int32)
    lanes = lax.iota(jnp.int32, 16)
    lo8 = lanes < 8
    lov = one16i * lo
    hiv = lov + RANGE

    pltpu.sync_copy(adt_hbm.at[pl.ds(lo, RANGE)], adl.at[pl.ds(0, RANGE)])
    for r in range(8):
        adl[RANGE + r, :] = z16

    def _zacc(i, c):
        for q in range(ROWW // 16):
            acc[i, pl.ds(q * 16, 16)] = z16
        return c
    lax.fori_loop(0, RANGE + 8, _zacc, 0)
    fbuf[pl.ds(16, 16)] = z16i
    pend[pl.ds(0, 16)] = z16i
    scnt[0] = 0

    def _fold_count(m01):
        fbuf[pl.ds(0, 16)] = m01
        v = m01 + fbuf[pl.ds(8, 16)]
        fbuf[pl.ds(0, 16)] = v
        v = v + fbuf[pl.ds(4, 16)]
        fbuf[pl.ds(0, 16)] = v
        v = v + fbuf[pl.ds(2, 16)]
        fbuf[pl.ds(0, 16)] = v
        v = v + fbuf[pl.ds(1, 16)]
        return v[0]

    def _process(par):
        def _edge(i, c):
            d = sdl[par, i]
            arow = adl[d, :]
            e = rows2[par, i, pl.ds(64, 16)] + arow
            e = jnp.where(e >= 0.0, e, 0.2 * e)
            ex = jnp.exp(e)
            plsc.addupdate(acc.at[d, pl.ds(64, 16)], jnp.where(lo8, ex, z16))
            for q in range(4):
                plsc.addupdate(acc.at[d, pl.ds(q * 16, 16)],
                               rows2[par, i, pl.ds(q * 16, 16)] * ex)
            return c
        lax.fori_loop(0, 16, _edge, 0)

    def _append(s, dloc):
        cnt = scnt[0]
        k = lax.rem(cnt, 16)
        pend[pl.ds(0, 16)] = jnp.where(lanes == one16i * k,
                                       one16i * s, pend[pl.ds(0, 16)])
        kb = lax.div(cnt, 16)
        par = lax.rem(kb, 2)
        sdl[par, k] = dloc
        scnt[0] = cnt + 1

        @pl.when(k == 15)
        def _flush():
            gidx[par, pl.ds(0, 16)] = pend[pl.ds(0, 16)]
            pltpu.async_copy(tab_hbm.at[gidx.at[par]], rows2.at[par], sem)
            pend[pl.ds(0, 16)] = z16i

            @pl.when(kb >= 1)
            def _():
                prev = 1 - par
                pltpu.make_async_copy(tab_hbm.at[gidx.at[prev]],
                                      rows2.at[prev], sem).wait()
                _process(prev)

    def _chunk(c, carry):
        off = c * ECH
        pltpu.sync_copy(dst_hbm.at[pl.ds(off, ECH)], dch)
        pltpu.sync_copy(src_hbm.at[pl.ds(off, ECH)], sch)

        def _group(g, cc):
            dvec = dch[pl.ds(g * 16, 16)]
            inr = jnp.logical_and(dvec >= lov, dvec < hiv)
            pc = _fold_count(jnp.where(inr, one16i, z16i))

            @pl.when(pc > 0)
            def _():
                svec = sch[pl.ds(g * 16, 16)]
                for i in range(16):
                    d = dvec[i]

                    @pl.when(jnp.logical_and(d >= lo, d < lo + RANGE))
                    def _(i=i, d=d, svec=svec):
                        _append(svec[i], d - lo)
            return cc

        lax.fori_loop(0, ECH // 16, _group, 0)
        return carry

    lax.fori_loop(0, NCHUNK, _chunk, 0)

    rem = lax.rem(scnt[0], 16)

    @pl.when(rem > 0)
    def _():
        def _pad(t, c):
            _append(0, TRASH)
            return c
        lax.fori_loop(0, 16 - rem, _pad, 0)

    @pl.when(scnt[0] >= 16)
    def _():
        last = lax.rem(lax.div(scnt[0], 16) - 1, 2)
        pltpu.make_async_copy(tab_hbm.at[gidx.at[last]],
                              rows2.at[last], sem).wait()
        _process(last)

    pltpu.sync_copy(acc.at[pl.ds(0, RANGE)], out_hbm.at[pl.ds(lo, RANGE)])


def kernel(x, edge_index, W1, a_src1, a_dst1, b1, W2, a_src2, a_dst2, b2):
    W1r = W1.reshape(IN, HEADS, HID)
    aw1s = jnp.einsum('ikf,kf->ik', W1r, a_src1)
    aw1d = jnp.einsum('ikf,kf->ik', W1r, a_dst1)
    zpad1 = jnp.zeros((IN, TABW - ROWW), jnp.float32)
    tab1w = jnp.concatenate([W1[:, _PERM], aw1s, aw1s, zpad1], 1)
    adst1w = jnp.concatenate([aw1d, aw1d], 1)
    W2p = W2[_PERM]
    W2r = W2p.reshape(64, HEADS, OUT)
    aw2s = jnp.einsum('ikf,kf->ik', W2r, a_src2)
    aw2d = jnp.einsum('ikf,kf->ik', W2r, a_dst2)
    zpad2 = jnp.zeros((64, TABW - ROWW), jnp.float32)
    tab2w = jnp.concatenate([W2p[:, _PERM], aw2s, aw2s, zpad2], 1)
    adst2w = jnp.concatenate([aw2d, aw2d], 1)
    b1t = b1[_PERM][None, :]
    b2r = b2[None, :]
    selk = jnp.tile(jnp.eye(8, dtype=jnp.float32), (1, 8))
    m3 = (jnp.kron(jnp.eye(8), jnp.ones((8, 1))) / 8.0).astype(jnp.float32)

    loop = jnp.arange(N, dtype=jnp.int32)
    padi = jnp.full((EPAD - ETOT,), N, jnp.int32)
    srcf = jnp.concatenate([edge_index[0], loop, padi])
    dstf = jnp.concatenate([edge_index[1], loop, padi])
    xp = jnp.pad(x, ((0, NPAD - N), (0, 0)))

    tab1, adt1 = _tc_prep(xp, tab1w, adst1w)
    acc1 = _edge_pass(tab1, adt1, srcf, dstf)
    tab2, adt2 = _tc_mid(acc1, b1t, selk, tab2w, adst2w)
    acc2 = _edge_pass(tab2, adt2, srcf, dstf)
    out = _tc_fin(acc2, b2r, selk, m3)
    return out[:N]
